# baseline (speedup 1.0000x reference)
import functools

import jax
import jax.numpy as jnp
from jax import lax
from jax.experimental import pallas as pl
from jax.experimental.pallas import tpu as pltpu
from jax.experimental.pallas import tpu_sc as plsc

EMBED = 64
NC = 2
NS = 16
NW = NC * NS
CHUNK = 128
NBUF = 4
NGB = 2
LANES = 16
SEQP = 16


@functools.lru_cache(maxsize=None)
def _build(batch: int, seq: int, vocab: int):
    mesh = plsc.VectorSubcoreMesh(core_axis_name="c", subcore_axis_name="s")
    rows_per_w = batch // NW
    n_q = rows_per_w // CHUNK
    n_tasks = seq * n_q
    stage_tiles = 8
    assert vocab % stage_tiles == 0 and rows_per_w % CHUNK == 0
    stage_rows = vocab // stage_tiles

    @functools.partial(
        pl.kernel,
        mesh=mesh,
        out_type=jax.ShapeDtypeStruct((seq, EMBED, batch), jnp.float32),
        scratch_types=[
            pltpu.VMEM((n_q * SEQP, CHUNK), jnp.int32),
            pltpu.VMEM((NGB, CHUNK, EMBED), jnp.float32),
            pltpu.VMEM((NBUF, EMBED, CHUNK), jnp.float32),
            pltpu.VMEM((stage_rows, EMBED), jnp.float32),
            pltpu.VMEM_SHARED((vocab, EMBED), jnp.float32),
            pltpu.SemaphoreType.DMA((NGB,)),
            pltpu.SemaphoreType.DMA((NBUF,)),
        ],
        compiler_params=pltpu.CompilerParams(use_tc_tiling_on_sc=False,
                                             needs_layout_passes=False),
    )
    def emb(idxT_hbm, table_hbm, out_hbm, idx_v, rows_v, tbuf, tv, table_sh,
            gsem, osem):
        sid = lax.axis_index("s")
        wid = sid * NC + lax.axis_index("c")
        b0 = wid * rows_per_w

        @pl.when(sid < stage_tiles)
        def _stage():
            lo = sid * stage_rows
            pltpu.sync_copy(
                table_hbm.at[pl.ds(lo, stage_rows), pl.ds(0, EMBED)], tv)
            pltpu.sync_copy(tv, table_sh.at[pl.ds(lo, stage_rows)])

        for q in range(n_q):
            pltpu.sync_copy(
                idxT_hbm.at[pl.ds(0, SEQP), pl.ds(b0 + q * CHUNK, CHUNK)],
                idx_v.at[pl.ds(q * SEQP, SEQP)])
        plsc.subcore_barrier()

        lane = lax.iota(jnp.int32, LANES)
        row16 = [lane + g * LANES for g in range(CHUNK // LANES)]

        def fire_gather(t, rb):
            s, q = t // n_q, t % n_q
            pltpu.async_copy(table_sh.at[idx_v.at[q * SEQP + s]],
                             rows_v.at[rb], gsem.at[rb])

        def wait_gather(rb):
            pltpu.make_async_copy(
                table_hbm.at[pl.ds(0, CHUNK), pl.ds(0, EMBED)],
                rows_v.at[rb], gsem.at[rb]).wait()

        def out_slice(t):
            s, q = t // n_q, t % n_q
            return out_hbm.at[s, :, pl.ds(b0 + q * CHUNK, CHUNK)]

        def fire_out(t, tb):
            pltpu.async_copy(tbuf.at[tb], out_slice(t), osem.at[tb])

        def wait_out(t, tb):
            pltpu.make_async_copy(tbuf.at[tb], out_slice(t),
                                  osem.at[tb]).wait()

        def transpose(rb, tb):
            def body(i, carry):
                d = i * 4
                dc = [lax.rem(lane + d + j, jnp.int32(LANES))
                      for j in range(4)]
                for h in range(EMBED // LANES):
                    ec = [c + h * LANES for c in dc]
                    for g in range(CHUNK // LANES):
                        vs = [plsc.load_gather(rows_v.at[rb], [row16[g], e])
                              for e in ec]
                        for e, v in zip(ec, vs):
                            plsc.store_scatter(tbuf.at[tb], [e, row16[g]], v)
                return carry

            lax.fori_loop(0, LANES // 4, body, 0)

        def fire_gather_d(s, q, rb):
            pltpu.async_copy(table_sh.at[idx_v.at[q * SEQP + s]],
                             rows_v.at[rb], gsem.at[rb])

        def out_slice_d(s, q):
            return out_hbm.at[s, :, pl.ds(b0 + q * CHUNK, CHUNK)]

        def plane(s, carry):
            for i in range(n_q):
                rb = i % NGB
                wait_gather(rb)
                if i == n_q - 1:
                    @pl.when(s < seq - 1)
                    def _fire():
                        fire_gather_d(s + 1, 0, 0)
                else:
                    fire_gather_d(s, i + 1, (i + 1) % NGB)

                @pl.when(s > 0)
                def _drain():
                    pltpu.make_async_copy(tbuf.at[i], out_slice_d(s - 1, i),
                                          osem.at[i]).wait()

                transpose(rb, i)
                pltpu.async_copy(tbuf.at[i], out_slice_d(s, i), osem.at[i])
            return carry

        fire_gather_d(0, 0, 0)
        lax.fori_loop(0, seq, plane, 0)
        for i in range(n_q):
            pltpu.make_async_copy(tbuf.at[i], out_slice_d(seq - 1, i),
                                  osem.at[i]).wait()

    return emb


def kernel(inputs, table):
    batch, seq = inputs.shape
    idxT = jnp.pad(inputs.astype(jnp.int32).T, ((0, SEQP - seq), (0, 0)))
    table_pad = jnp.pad(table, ((0, 0), (0, CHUNK - EMBED)))
    out = _build(batch, seq, table.shape[0])(idxT, table_pad)
    return out.transpose(2, 0, 1)


# speedup vs baseline: 1.0112x; 1.0112x over previous
import functools

import jax
import jax.numpy as jnp
from jax import lax
from jax.experimental import pallas as pl
from jax.experimental.pallas import tpu as pltpu
from jax.experimental.pallas import tpu_sc as plsc

EMBED = 64
NC = 2
NS = 16
NW = NC * NS
CHUNK = 128
NBUF = 4
NGB = 2
LANES = 16
SEQP = 16


@functools.lru_cache(maxsize=None)
def _build(batch: int, seq: int, vocab: int):
    mesh = plsc.VectorSubcoreMesh(core_axis_name="c", subcore_axis_name="s")
    rows_per_w = batch // NW
    n_q = rows_per_w // CHUNK
    n_tasks = seq * n_q
    stage_tiles = 8
    assert vocab % stage_tiles == 0 and rows_per_w % CHUNK == 0
    stage_rows = vocab // stage_tiles

    @functools.partial(
        pl.kernel,
        mesh=mesh,
        out_type=jax.ShapeDtypeStruct((seq, EMBED, batch), jnp.float32),
        scratch_types=[
            pltpu.VMEM((n_q * SEQP, CHUNK), jnp.int32),
            pltpu.VMEM((NGB, CHUNK, EMBED), jnp.float32),
            pltpu.VMEM((NBUF, EMBED, CHUNK), jnp.float32),
            pltpu.VMEM((stage_rows, EMBED), jnp.float32),
            pltpu.VMEM_SHARED((vocab, EMBED), jnp.float32),
            pltpu.SemaphoreType.DMA((NGB,)),
            pltpu.SemaphoreType.DMA((NBUF,)),
        ],
        compiler_params=pltpu.CompilerParams(use_tc_tiling_on_sc=False,
                                             needs_layout_passes=False),
    )
    def emb(idxT_hbm, table_hbm, out_hbm, idx_v, rows_v, tbuf, tv, table_sh,
            gsem, osem):
        sid = lax.axis_index("s")
        wid = sid * NC + lax.axis_index("c")
        b0 = wid * rows_per_w

        @pl.when(sid < stage_tiles)
        def _stage():
            lo = sid * stage_rows
            pltpu.sync_copy(
                table_hbm.at[pl.ds(lo, stage_rows), pl.ds(0, EMBED)], tv)
            pltpu.sync_copy(tv, table_sh.at[pl.ds(lo, stage_rows)])

        for q in range(n_q):
            pltpu.sync_copy(
                idxT_hbm.at[pl.ds(0, SEQP), pl.ds(b0 + q * CHUNK, CHUNK)],
                idx_v.at[pl.ds(q * SEQP, SEQP)])
        plsc.subcore_barrier()

        lane = lax.iota(jnp.int32, LANES)
        row16 = [lane + g * LANES for g in range(CHUNK // LANES)]

        def fire_gather(t, rb):
            s, q = t // n_q, t % n_q
            pltpu.async_copy(table_sh.at[idx_v.at[q * SEQP + s]],
                             rows_v.at[rb], gsem.at[rb])

        def wait_gather(rb):
            pltpu.make_async_copy(
                table_hbm.at[pl.ds(0, CHUNK), pl.ds(0, EMBED)],
                rows_v.at[rb], gsem.at[rb]).wait()

        def out_slice(t):
            s, q = t // n_q, t % n_q
            return out_hbm.at[s, :, pl.ds(b0 + q * CHUNK, CHUNK)]

        def fire_out(t, tb):
            pltpu.async_copy(tbuf.at[tb], out_slice(t), osem.at[tb])

        def wait_out(t, tb):
            pltpu.make_async_copy(tbuf.at[tb], out_slice(t),
                                  osem.at[tb]).wait()

        def transpose(rb, tb):
            def body(i, carry):
                d = i * 2
                da = lax.rem(lane + d, jnp.int32(LANES))
                db = lax.rem(lane + d + 1, jnp.int32(LANES))
                for h in range(EMBED // LANES):
                    ea = da + h * LANES
                    eb = db + h * LANES
                    for g in range(CHUNK // LANES):
                        va = plsc.load_gather(rows_v.at[rb], [row16[g], ea])
                        vb = plsc.load_gather(rows_v.at[rb], [row16[g], eb])
                        plsc.store_scatter(tbuf.at[tb], [ea, row16[g]], va)
                        plsc.store_scatter(tbuf.at[tb], [eb, row16[g]], vb)
                return carry

            lax.fori_loop(0, LANES // 2, body, 0)

        def fire_gather_d(s, q, rb):
            pltpu.async_copy(table_sh.at[idx_v.at[q * SEQP + s]],
                             rows_v.at[rb], gsem.at[rb])

        def out_slice_d(s, q):
            return out_hbm.at[s, :, pl.ds(b0 + q * CHUNK, CHUNK)]

        def plane(s, first, last):
            for i in range(n_q):
                rb = i % NGB
                wait_gather(rb)
                if not (last and i == n_q - 1):
                    nq = (i + 1) % n_q
                    ns = s + (i + 1) // n_q
                    fire_gather_d(ns, nq, (i + 1) % NGB)
                if not first:
                    pltpu.make_async_copy(tbuf.at[i], out_slice_d(s - 1, i),
                                          osem.at[i]).wait()
                transpose(rb, i)
                pltpu.async_copy(tbuf.at[i], out_slice_d(s, i), osem.at[i])

        fire_gather_d(0, 0, 0)
        plane(0, first=True, last=False)

        def mid(s, carry):
            plane(s, first=False, last=False)
            return carry

        lax.fori_loop(1, seq - 1, mid, 0)
        plane(seq - 1, first=False, last=True)
        for i in range(n_q):
            pltpu.make_async_copy(tbuf.at[i], out_slice_d(seq - 1, i),
                                  osem.at[i]).wait()

    return emb


def kernel(inputs, table):
    batch, seq = inputs.shape
    idxT = jnp.pad(inputs.astype(jnp.int32).T, ((0, SEQP - seq), (0, 0)))
    table_pad = jnp.pad(table, ((0, 0), (0, CHUNK - EMBED)))
    out = _build(batch, seq, table.shape[0])(idxT, table_pad)
    return out.transpose(2, 0, 1)
